# baseline (speedup 1.0000x reference)
import functools

import jax
import jax.numpy as jnp
from jax import lax
from jax.experimental import pallas as pl
from jax.experimental.pallas import tpu as pltpu
from jax.experimental.pallas import tpu_sc as plsc

N = 50000
D = 128
S = 10
EMB = 128
B = 8192

_INFO = plsc.get_sparse_core_info()
_NC = _INFO.num_cores
_NS = _INFO.num_subcores
_NW = _NC * _NS
_B_PER_W = B // _NW
_CHUNK = 128
_NCHUNK = _B_PER_W // _CHUNK


def _sc_self_kernel(feat_hbm, nodes_hbm, self_out, tok_out,
                    nodes_v, self0_v, self1_v, sem_g, sem_w):
  self_v = (self0_v, self1_v)
  wid = lax.axis_index("s") * _NC + lax.axis_index("c")
  pltpu.sync_copy(nodes_hbm.at[pl.ds(wid * _B_PER_W, _B_PER_W)], nodes_v)
  cps = [pltpu.async_copy(feat_hbm.at[nodes_v.at[pl.ds(c * _CHUNK, _CHUNK)]],
                          self_v[c], sem_g)
         for c in range(_NCHUNK)]
  @pl.when(wid == 0)
  def _():
    pltpu.sync_copy(nodes_v.at[pl.ds(0, 128)], tok_out.at[0])
  outs = []
  for c in range(_NCHUNK):
    cps[c].wait()
    outs.append(pltpu.async_copy(
        self_v[c], self_out.at[pl.ds(wid * _B_PER_W + c * _CHUNK, _CHUNK)],
        sem_w))
  for cp in outs:
    cp.wait()


def _sc_gather_kernel(feat_hbm, nb_hbm, nsum_out,
                      nb0_v, nb1_v, nbt0_v, nbt1_v,
                      nsum0_v, nsum1_v,
                      sem_nb0, sem_nb1,
                      sem_s00, sem_s01, sem_acc0, sem_acc1, sem_out):
  nb_v = (nb0_v, nb1_v)
  nbt_v = (nbt0_v, nbt1_v)
  nsum_v = (nsum0_v, nsum1_v)
  sem_nb = (sem_nb0, sem_nb1)
  sem_s0 = (sem_s00, sem_s01)
  sem_acc = (sem_acc0, sem_acc1)
  wid = lax.axis_index("s") * _NC + lax.axis_index("c")
  cp_nb = [pltpu.async_copy(
      nb_hbm.at[pl.ds(wid * _B_PER_W + c * _CHUNK, _CHUNK)], nb_v[c],
      sem_nb[c]) for c in range(_NCHUNK)]
  cps_s0 = []
  for c in range(_NCHUNK):
    cp_nb[c].wait()
    for s in range(S):
      col = jnp.full((16,), s, jnp.int32)
      for g in range(_CHUNK // 16):
        rows = lax.iota(jnp.int32, 16) + jnp.int32(g * 16)
        nbt_v[c][s, pl.ds(g * 16, 16)] = plsc.load_gather(nb_v[c],
                                                          [rows, col])
    cps_s0.append(pltpu.async_copy(feat_hbm.at[nbt_v[c].at[0]], nsum_v[c],
                                   sem_s0[c]))
  cps_acc = []
  for c in range(_NCHUNK):
    cps_s0[c].wait()
    cps_acc.append([pltpu.async_copy(feat_hbm.at[nbt_v[c].at[s]], nsum_v[c],
                                     sem_acc[c], add=True)
                    for s in range(1, S)])
  cps_out = []
  for c in range(_NCHUNK):
    for cp in cps_acc[c]:
      cp.wait()
    base = (wid * _B_PER_W) + c * _CHUNK
    cps_out.append(pltpu.async_copy(
        nsum_v[c], nsum_out.at[pl.ds(base, _CHUNK)], sem_out))
  for cp in cps_out:
    cp.wait()


def _tc_matmul_kernel(x_ref, n_ref, ws_ref, wn_ref, o_ref):
  acc = jnp.dot(x_ref[...], ws_ref[...], preferred_element_type=jnp.float32)
  acc += jnp.dot(n_ref[...], wn_ref[...], preferred_element_type=jnp.float32)
  o_ref[...] = jnp.maximum(acc, 0.0)


_BM = 2048


@jax.jit
def kernel(nodes, feat_data, neigh_idx, W):
  mesh = plsc.VectorSubcoreMesh(core_axis_name="c", subcore_axis_name="s")

  sc_self = pl.kernel(
      _sc_self_kernel,
      out_type=(jax.ShapeDtypeStruct((B, D), jnp.float32),
                jax.ShapeDtypeStruct((8, 128), jnp.int32)),
      mesh=mesh,
      scratch_types=[
          pltpu.VMEM((_B_PER_W,), jnp.int32),
          pltpu.VMEM((_CHUNK, D), jnp.float32),
          pltpu.VMEM((_CHUNK, D), jnp.float32),
          pltpu.SemaphoreType.DMA,
          pltpu.SemaphoreType.DMA,
      ],
  )
  self_feats, tok_arr = sc_self(feat_data, nodes)

  tok = tok_arr[0, 0] * jnp.int32(0)
  nb = neigh_idx.at[nodes + tok].get(mode="promise_in_bounds")

  sc_gather = pl.kernel(
      _sc_gather_kernel,
      out_type=jax.ShapeDtypeStruct((B, D), jnp.float32),
      mesh=mesh,
      scratch_types=[
          pltpu.VMEM((_CHUNK, S), jnp.int32),
          pltpu.VMEM((_CHUNK, S), jnp.int32),
          pltpu.VMEM((S, _CHUNK), jnp.int32),
          pltpu.VMEM((S, _CHUNK), jnp.int32),
          pltpu.VMEM((_CHUNK, D), jnp.float32),
          pltpu.VMEM((_CHUNK, D), jnp.float32),
      ] + [pltpu.SemaphoreType.DMA] * 7,
      compiler_params=pltpu.CompilerParams(needs_layout_passes=False),
  )
  nsum = sc_gather(feat_data, nb)

  ws = W[:, :D].T
  wn = W[:, D:].T * jnp.float32(1.0 / S)
  out = pl.pallas_call(
      _tc_matmul_kernel,
      grid=(B // _BM,),
      in_specs=[
          pl.BlockSpec((_BM, D), lambda i: (i, 0)),
          pl.BlockSpec((_BM, D), lambda i: (i, 0)),
          pl.BlockSpec((D, EMB), lambda i: (0, 0)),
          pl.BlockSpec((D, EMB), lambda i: (0, 0)),
      ],
      out_specs=pl.BlockSpec((_BM, EMB), lambda i: (i, 0)),
      out_shape=jax.ShapeDtypeStruct((B, EMB), jnp.float32),
  )(self_feats, nsum, ws, wn)
  return out


# speedup vs baseline: 1.0644x; 1.0644x over previous
import jax
import jax.numpy as jnp
from jax import lax
from jax.experimental import pallas as pl
from jax.experimental.pallas import tpu as pltpu
from jax.experimental.pallas import tpu_sc as plsc

N = 50000
D = 128
S = 10
EMB = 128
B = 8192

_INFO = plsc.get_sparse_core_info()
_NC = _INFO.num_cores
_NS = _INFO.num_subcores
_NW = _NC * _NS
_B_PER_W = B // _NW
_CHUNK = 128
_NCHUNK = _B_PER_W // _CHUNK


def _sc_gather_kernel(feat_hbm, nodes_hbm, nb_hbm, self_out, nsum_out,
                      nodes_v, nb0_v, nb1_v, nbt0_v, nbt1_v,
                      self0_v, self1_v, nsum0_v, nsum1_v,
                      sem_self0, sem_self1, sem_nb0, sem_nb1,
                      sem_s00, sem_s01, sem_acc0, sem_acc1, sem_out):
  nb_v = (nb0_v, nb1_v)
  nbt_v = (nbt0_v, nbt1_v)
  self_v = (self0_v, self1_v)
  nsum_v = (nsum0_v, nsum1_v)
  sem_self = (sem_self0, sem_self1)
  sem_nb = (sem_nb0, sem_nb1)
  sem_s0 = (sem_s00, sem_s01)
  sem_acc = (sem_acc0, sem_acc1)
  wid = lax.axis_index("s") * _NC + lax.axis_index("c")
  pltpu.sync_copy(nodes_hbm.at[pl.ds(wid * _NCHUNK, _NCHUNK)], nodes_v)
  cp_nb = [pltpu.async_copy(
      nb_hbm.at[pl.ds(wid * _B_PER_W + c * _CHUNK, _CHUNK)], nb_v[c],
      sem_nb[c]) for c in range(_NCHUNK)]
  cp_self = [pltpu.async_copy(feat_hbm.at[nodes_v.at[c]], self_v[c],
                              sem_self[c]) for c in range(_NCHUNK)]
  cps_s0 = []
  for c in range(_NCHUNK):
    cp_nb[c].wait()
    for s in range(S):
      col = jnp.full((16,), s, jnp.int32)
      for g in range(_CHUNK // 16):
        rows = lax.iota(jnp.int32, 16) + jnp.int32(g * 16)
        nbt_v[c][s, pl.ds(g * 16, 16)] = plsc.load_gather(nb_v[c],
                                                          [rows, col])
    cps_s0.append(pltpu.async_copy(feat_hbm.at[nbt_v[c].at[0]], nsum_v[c],
                                   sem_s0[c]))
  cps_acc = []
  for c in range(_NCHUNK):
    cps_s0[c].wait()
    cps_acc.append([pltpu.async_copy(feat_hbm.at[nbt_v[c].at[s]], nsum_v[c],
                                     sem_acc[c], add=True)
                    for s in range(1, S)])
  cps_out = []
  for c in range(_NCHUNK):
    for cp in cps_acc[c]:
      cp.wait()
    cp_self[c].wait()
    base = (wid * _B_PER_W) + c * _CHUNK
    cps_out.append(pltpu.async_copy(
        self_v[c], self_out.at[pl.ds(base, _CHUNK)], sem_out))
    cps_out.append(pltpu.async_copy(
        nsum_v[c], nsum_out.at[pl.ds(base, _CHUNK)], sem_out))
  for cp in cps_out:
    cp.wait()


def _tc_matmul_kernel(x_ref, n_ref, ws_ref, wn_ref, o_ref):
  acc = jnp.dot(x_ref[...], ws_ref[...], preferred_element_type=jnp.float32)
  acc += jnp.dot(n_ref[...], wn_ref[...], preferred_element_type=jnp.float32)
  o_ref[...] = jnp.maximum(acc, 0.0)


_BM = 2048


@jax.jit
def kernel(nodes, feat_data, neigh_idx, W):
  nodes2 = nodes.reshape(B // 128, 128)

  nb = neigh_idx.at[nodes].get(mode="promise_in_bounds")

  mesh = plsc.VectorSubcoreMesh(core_axis_name="c", subcore_axis_name="s")
  sc_gather = pl.kernel(
      _sc_gather_kernel,
      out_type=(jax.ShapeDtypeStruct((B, D), jnp.float32),
                jax.ShapeDtypeStruct((B, D), jnp.float32)),
      mesh=mesh,
      scratch_types=[
          pltpu.VMEM((_NCHUNK, _CHUNK), jnp.int32),
          pltpu.VMEM((_CHUNK, S), jnp.int32),
          pltpu.VMEM((_CHUNK, S), jnp.int32),
          pltpu.VMEM((S, _CHUNK), jnp.int32),
          pltpu.VMEM((S, _CHUNK), jnp.int32),
          pltpu.VMEM((_CHUNK, D), jnp.float32),
          pltpu.VMEM((_CHUNK, D), jnp.float32),
          pltpu.VMEM((_CHUNK, D), jnp.float32),
          pltpu.VMEM((_CHUNK, D), jnp.float32),
      ] + [pltpu.SemaphoreType.DMA] * 9,
      compiler_params=pltpu.CompilerParams(needs_layout_passes=False),
  )
  self_feats, nsum = sc_gather(feat_data, nodes2, nb)

  ws = W[:, :D].T
  wn = W[:, D:].T * jnp.float32(1.0 / S)
  out = pl.pallas_call(
      _tc_matmul_kernel,
      grid=(B // _BM,),
      in_specs=[
          pl.BlockSpec((_BM, D), lambda i: (i, 0)),
          pl.BlockSpec((_BM, D), lambda i: (i, 0)),
          pl.BlockSpec((D, EMB), lambda i: (0, 0)),
          pl.BlockSpec((D, EMB), lambda i: (0, 0)),
      ],
      out_specs=pl.BlockSpec((_BM, EMB), lambda i: (i, 0)),
      out_shape=jax.ShapeDtypeStruct((B, EMB), jnp.float32),
  )(self_feats, nsum, ws, wn)
  return out
